# baseline (speedup 1.0000x reference)
import functools

import jax
import jax.numpy as jnp
from jax import lax
from jax.experimental import pallas as pl
from jax.experimental.pallas import tpu as pltpu
from jax.experimental.pallas import tpu_sc as plsc

N = 10000
E = 320000
NC = 2
NS = 16
NW = NC * NS
NPAD = 10240
CH = 128

_MESH = dict(core_axis_name="c", subcore_axis_name="s",
             num_cores=NC, num_subcores=NS)


def _deg_hist(dst):
  epw = E // NW
  nfull = epw // CH
  tail = epw - nfull * CH
  rpw = NPAD // NS

  @functools.partial(
      pl.kernel,
      out_type=jax.ShapeDtypeStruct((NC, NPAD), jnp.float32),
      mesh=plsc.VectorSubcoreMesh(**_MESH),
      scratch_types=[
          pltpu.VMEM((CH,), jnp.int32),
          pltpu.VMEM((CH,), jnp.int32),
          pltpu.VMEM((tail,), jnp.int32),
          pltpu.VMEM((CH,), jnp.float32),
          pltpu.VMEM((rpw,), jnp.float32),
          pltpu.VMEM_SHARED((NPAD,), jnp.float32),
          pltpu.SemaphoreType.DMA,
          pltpu.SemaphoreType.DMA,
      ],
  )
  def hist(dst_hbm, out_hbm, ib0, ib1, ibt, ones_v, zbuf, acc, sem0, sem1):
    c = lax.axis_index("c")
    s = lax.axis_index("s")
    wid = c * NS + s
    one16 = jnp.ones((16,), jnp.float32)
    zero16 = jnp.zeros((16,), jnp.float32)
    for k in range(CH // 16):
      ones_v[pl.ds(k * 16, 16)] = one16

    @pl.loop(0, rpw // 16)
    def _(i):
      zbuf[pl.ds(i * 16, 16)] = zero16

    pltpu.sync_copy(zbuf, acc.at[pl.ds(s * rpw, rpw)])
    plsc.subcore_barrier()

    def idx_base(k):
      kc = jnp.minimum(k, nfull - 1)
      return pl.multiple_of(wid * epw + kc * CH, 8)

    def start_idx(k, ib, sem):
      pltpu.async_copy(dst_hbm.at[pl.ds(idx_base(k), CH)], ib, sem)

    def wait_idx(ib, sem):
      pltpu.make_async_copy(dst_hbm.at[pl.ds(0, CH)], ib, sem).wait()

    def scatter(ib):
      pltpu.sync_copy(ones_v, acc.at[ib], add=True)

    pltpu.sync_copy(dst_hbm.at[pl.ds(idx_base(0), CH)], ib0)
    start_idx(1, ib1, sem1)

    @pl.loop(0, nfull // 2)
    def _(p):
      j = p * 2
      scatter(ib0)
      start_idx(j + 2, ib0, sem0)
      wait_idx(ib1, sem1)
      scatter(ib1)
      start_idx(j + 3, ib1, sem1)
      wait_idx(ib0, sem0)

    wait_idx(ib1, sem1)
    tbase = pl.multiple_of(wid * epw + nfull * CH, 8)
    pltpu.sync_copy(dst_hbm.at[pl.ds(tbase, tail)], ibt)
    pltpu.sync_copy(ones_v.at[pl.ds(0, tail)], acc.at[ibt], add=True)
    plsc.subcore_barrier()
    pltpu.sync_copy(acc.at[pl.ds(s * rpw, rpw)], out_hbm.at[c, pl.ds(s * rpw, rpw)])

  return hist(dst)


def _edge_agg(hp, eidx, width):
  epw = E // NW
  nfull = epw // CH
  tail = epw - nfull * CH
  rpw = NPAD // NS

  @functools.partial(
      pl.kernel,
      out_type=jax.ShapeDtypeStruct((NC, NPAD, width), jnp.float32),
      mesh=plsc.VectorSubcoreMesh(**_MESH),
      scratch_types=[
          [pltpu.VMEM((2, CH), jnp.int32)] * 4,
          pltpu.VMEM((2, tail), jnp.int32),
          [pltpu.VMEM((CH, width), jnp.float32)] * 4,
          pltpu.VMEM((tail, width), jnp.float32),
          pltpu.VMEM_SHARED((NPAD, width), jnp.float32),
          [pltpu.SemaphoreType.DMA] * 4,
          [pltpu.SemaphoreType.DMA] * 4,
          [pltpu.SemaphoreType.DMA] * 4,
      ],
      compiler_params=pltpu.CompilerParams(use_tc_tiling_on_sc=False),
  )
  def agg(hp_hbm, eidx_hbm, out_hbm, ebs, ebt, rbs, rbt, acc,
          semis, semgs, semss):
    c = lax.axis_index("c")
    s = lax.axis_index("s")
    wid = c * NS + s
    zero16 = jnp.zeros((16,), jnp.float32)
    rb0 = rbs[0]

    @pl.loop(0, CH)
    def _(i):
      for k in range(width // 16):
        rb0[i, pl.ds(k * 16, 16)] = zero16

    for j in range(rpw // CH):
      pltpu.sync_copy(rb0, acc.at[pl.ds(s * rpw + j * CH, CH), :])
    plsc.subcore_barrier()

    def idx_base(k):
      kc = jnp.minimum(k, nfull - 1)
      return pl.multiple_of(wid * epw + kc * CH, 8)

    def start_idx(k, b):
      pltpu.async_copy(eidx_hbm.at[:, pl.ds(idx_base(k), CH)], ebs[b], semis[b])

    def wait_idx(b):
      pltpu.make_async_copy(eidx_hbm.at[:, pl.ds(0, CH)], ebs[b], semis[b]).wait()

    def start_gather(b):
      pltpu.async_copy(hp_hbm.at[ebs[b].at[0]], rbs[b], semgs[b])

    def wait_gather(b):
      pltpu.make_async_copy(hp_hbm.at[ebs[b].at[0]], rbs[b], semgs[b]).wait()

    def start_scatter(b):
      pltpu.async_copy(rbs[b], acc.at[ebs[b].at[1]], semss[b], add=True)

    def wait_scatter(b):
      pltpu.make_async_copy(rbs[b], acc.at[ebs[b].at[1]], semss[b]).wait()

    def step(k, bk, skip_scatter_wait):
      b1 = (bk + 1) % 4
      b2 = (bk + 2) % 4
      wait_idx(b1)
      start_gather(b1)
      wait_gather(bk)
      start_scatter(bk)
      if not skip_scatter_wait:
        wait_scatter(b2)
      start_idx(k + 2, b2)

    pltpu.sync_copy(eidx_hbm.at[:, pl.ds(idx_base(0), CH)], ebs[0])
    start_gather(0)
    start_idx(1, 1)
    step(jnp.int32(0), 0, True)
    step(jnp.int32(1), 1, True)

    @pl.loop(0, (nfull - 2) // 4)
    def _(p):
      for m in range(4):
        step(2 + p * 4 + m, (2 + m) % 4, False)

    wait_idx((nfull + 1) % 4)
    wait_gather(nfull % 4)
    wait_scatter((nfull - 2) % 4)
    wait_scatter((nfull - 1) % 4)
    tbase = pl.multiple_of(wid * epw + nfull * CH, 8)
    pltpu.sync_copy(eidx_hbm.at[:, pl.ds(tbase, tail)], ebt)
    pltpu.async_copy(hp_hbm.at[ebt.at[0]], rbt, semgs[0]).wait()
    pltpu.sync_copy(rbt, acc.at[ebt.at[1]], add=True)
    plsc.subcore_barrier()
    pltpu.sync_copy(acc.at[pl.ds(s * rpw, rpw), :],
                    out_hbm.at[c, pl.ds(s * rpw, rpw), :])

  return agg(hp, eidx)


def _dinv_block(d_ref):
  deg = d_ref[:, 0:1] + d_ref[:, 1:2] + jnp.float32(1.0)
  return lax.rsqrt(deg)


def _tc_layer1(x, w1):
  b = 2000
  grid = N // b

  def body(x_ref, w_ref, o_ref):
    o_ref[...] = jnp.dot(x_ref[...], w_ref[...],
                         preferred_element_type=jnp.float32)

  return pl.pallas_call(
      body,
      grid=(grid,),
      in_specs=[
          pl.BlockSpec((b, 128), lambda i: (i, 0)),
          pl.BlockSpec((128, 64), lambda i: (0, 0)),
      ],
      out_specs=pl.BlockSpec((b, 64), lambda i: (i, 0)),
      out_shape=jax.ShapeDtypeStruct((NPAD, 64), jnp.float32),
  )(x, w1)


def _tc_scale(h1, dpt):
  b = 2000
  grid = N // b

  def body(h_ref, d_ref, o_ref):
    o_ref[...] = h_ref[...] * _dinv_block(d_ref)

  return pl.pallas_call(
      body,
      grid=(grid,),
      in_specs=[
          pl.BlockSpec((b, 64), lambda i: (i, 0)),
          pl.BlockSpec((b, 2), lambda i: (i, 0)),
      ],
      out_specs=pl.BlockSpec((b, 64), lambda i: (i, 0)),
      out_shape=jax.ShapeDtypeStruct((NPAD, 64), jnp.float32),
  )(h1, dpt)


def _tc_layer2(ap1, hp1, dpt, b1, w2):
  b = 2000
  grid = N // b

  def body(a_ref, h_ref, d_ref, b_ref, w_ref, o_ref):
    dinv = _dinv_block(d_ref)
    t = a_ref[0] + a_ref[1] + h_ref[...]
    x2 = jnp.maximum(t * dinv + b_ref[...], jnp.float32(0.0))
    o_ref[...] = jnp.dot(x2, w_ref[...], preferred_element_type=jnp.float32) * dinv

  return pl.pallas_call(
      body,
      grid=(grid,),
      in_specs=[
          pl.BlockSpec((NC, b, 64), lambda i: (0, i, 0)),
          pl.BlockSpec((b, 64), lambda i: (i, 0)),
          pl.BlockSpec((b, 2), lambda i: (i, 0)),
          pl.BlockSpec((1, 64), lambda i: (0, 0)),
          pl.BlockSpec((64, 16), lambda i: (0, 0)),
      ],
      out_specs=pl.BlockSpec((b, 16), lambda i: (i, 0)),
      out_shape=jax.ShapeDtypeStruct((NPAD, 16), jnp.float32),
  )(ap1, hp1, dpt, b1, w2)


def _tc_final(ap2, hp2, dpt, b2):
  b = 2000
  grid = N // b

  def body(a_ref, h_ref, d_ref, b_ref, o_ref):
    t = a_ref[0] + a_ref[1] + h_ref[...]
    o_ref[...] = t * _dinv_block(d_ref) + b_ref[...]

  return pl.pallas_call(
      body,
      grid=(grid,),
      in_specs=[
          pl.BlockSpec((NC, b, 16), lambda i: (0, i, 0)),
          pl.BlockSpec((b, 16), lambda i: (i, 0)),
          pl.BlockSpec((b, 2), lambda i: (i, 0)),
          pl.BlockSpec((1, 16), lambda i: (0, 0)),
      ],
      out_specs=pl.BlockSpec((b, 16), lambda i: (i, 0)),
      out_shape=jax.ShapeDtypeStruct((N, 16), jnp.float32),
  )(ap2, hp2, dpt, b2)


def kernel(X, edge_index, W1, b1, W2, b2):
  deg_parts = _deg_hist(edge_index[1])
  h1 = _tc_layer1(X, W1)
  dpt = jnp.transpose(deg_parts)
  hp1 = _tc_scale(h1, dpt)
  ap1 = _edge_agg(hp1, edge_index, 64)
  hp2 = _tc_layer2(ap1, hp1, dpt, b1.reshape(1, 64), W2)
  ap2 = _edge_agg(hp2, edge_index, 16)
  return _tc_final(ap2, hp2, dpt, b2.reshape(1, 16))


# speedup vs baseline: 1.0033x; 1.0033x over previous
import functools

import jax
import jax.numpy as jnp
from jax import lax
from jax.experimental import pallas as pl
from jax.experimental.pallas import tpu as pltpu
from jax.experimental.pallas import tpu_sc as plsc

N = 10000
E = 320000
NC = 2
NS = 16
NW = NC * NS
NPAD = 10240
CH = 128

_MESH = dict(core_axis_name="c", subcore_axis_name="s",
             num_cores=NC, num_subcores=NS)


def _deg_hist(dst):
  epw = E // NW
  nfull = epw // CH
  tail = epw - nfull * CH
  rpw = NPAD // NS

  @functools.partial(
      pl.kernel,
      out_type=jax.ShapeDtypeStruct((NC, NPAD), jnp.float32),
      mesh=plsc.VectorSubcoreMesh(**_MESH),
      scratch_types=[
          pltpu.VMEM((CH,), jnp.int32),
          pltpu.VMEM((CH,), jnp.int32),
          pltpu.VMEM((tail,), jnp.int32),
          pltpu.VMEM((CH,), jnp.float32),
          pltpu.VMEM((rpw,), jnp.float32),
          pltpu.VMEM_SHARED((NPAD,), jnp.float32),
          pltpu.SemaphoreType.DMA,
          pltpu.SemaphoreType.DMA,
      ],
  )
  def hist(dst_hbm, out_hbm, ib0, ib1, ibt, ones_v, zbuf, acc, sem0, sem1):
    c = lax.axis_index("c")
    s = lax.axis_index("s")
    wid = c * NS + s
    one16 = jnp.ones((16,), jnp.float32)
    zero16 = jnp.zeros((16,), jnp.float32)
    for k in range(CH // 16):
      ones_v[pl.ds(k * 16, 16)] = one16

    @pl.loop(0, rpw // 16)
    def _(i):
      zbuf[pl.ds(i * 16, 16)] = zero16

    pltpu.sync_copy(zbuf, acc.at[pl.ds(s * rpw, rpw)])
    plsc.subcore_barrier()

    def idx_base(k):
      kc = jnp.minimum(k, nfull - 1)
      return pl.multiple_of(wid * epw + kc * CH, 8)

    def start_idx(k, ib, sem):
      pltpu.async_copy(dst_hbm.at[pl.ds(idx_base(k), CH)], ib, sem)

    def wait_idx(ib, sem):
      pltpu.make_async_copy(dst_hbm.at[pl.ds(0, CH)], ib, sem).wait()

    def scatter(ib):
      pltpu.sync_copy(ones_v, acc.at[ib], add=True)

    pltpu.sync_copy(dst_hbm.at[pl.ds(idx_base(0), CH)], ib0)
    start_idx(1, ib1, sem1)

    @pl.loop(0, nfull // 2)
    def _(p):
      j = p * 2
      scatter(ib0)
      start_idx(j + 2, ib0, sem0)
      wait_idx(ib1, sem1)
      scatter(ib1)
      start_idx(j + 3, ib1, sem1)
      wait_idx(ib0, sem0)

    wait_idx(ib1, sem1)
    tbase = pl.multiple_of(wid * epw + nfull * CH, 8)
    pltpu.sync_copy(dst_hbm.at[pl.ds(tbase, tail)], ibt)
    pltpu.sync_copy(ones_v.at[pl.ds(0, tail)], acc.at[ibt], add=True)
    plsc.subcore_barrier()
    pltpu.sync_copy(acc.at[pl.ds(s * rpw, rpw)], out_hbm.at[c, pl.ds(s * rpw, rpw)])

  return hist(dst)


def _edge_agg(hp, eidx, width):
  epw = E // NW
  nfull = epw // CH
  tail = epw - nfull * CH
  rpw = NPAD // NS

  @functools.partial(
      pl.kernel,
      out_type=jax.ShapeDtypeStruct((NC, NPAD, width), jnp.float32),
      mesh=plsc.VectorSubcoreMesh(**_MESH),
      scratch_types=[
          [pltpu.VMEM((2, CH), jnp.int32)] * 4,
          pltpu.VMEM((2, tail), jnp.int32),
          [pltpu.VMEM((CH, width), jnp.float32)] * 4,
          pltpu.VMEM((tail, width), jnp.float32),
          pltpu.VMEM_SHARED((NPAD, width), jnp.float32),
          [pltpu.SemaphoreType.DMA] * 4,
          [pltpu.SemaphoreType.DMA] * 4,
          [pltpu.SemaphoreType.DMA] * 4,
      ],
      compiler_params=pltpu.CompilerParams(use_tc_tiling_on_sc=False),
  )
  def agg(hp_hbm, eidx_hbm, out_hbm, ebs, ebt, rbs, rbt, acc,
          semis, semgs, semss):
    c = lax.axis_index("c")
    s = lax.axis_index("s")
    wid = c * NS + s
    zero16 = jnp.zeros((16,), jnp.float32)
    rb0 = rbs[0]

    @pl.loop(0, CH)
    def _(i):
      for k in range(width // 16):
        rb0[i, pl.ds(k * 16, 16)] = zero16

    for j in range(rpw // CH):
      pltpu.sync_copy(rb0, acc.at[pl.ds(s * rpw + j * CH, CH), :])
    plsc.subcore_barrier()

    def idx_base(k):
      kc = jnp.minimum(k, nfull - 1)
      return pl.multiple_of(wid * epw + kc * CH, 8)

    def start_idx(k, b):
      pltpu.async_copy(eidx_hbm.at[:, pl.ds(idx_base(k), CH)], ebs[b], semis[b])

    def wait_idx(b):
      pltpu.make_async_copy(eidx_hbm.at[:, pl.ds(0, CH)], ebs[b], semis[b]).wait()

    def start_gather(b):
      pltpu.async_copy(hp_hbm.at[ebs[b].at[0]], rbs[b], semgs[b])

    def wait_gather(b):
      pltpu.make_async_copy(hp_hbm.at[ebs[b].at[0]], rbs[b], semgs[b]).wait()

    def start_scatter(b):
      pltpu.async_copy(rbs[b], acc.at[ebs[b].at[1]], semss[b], add=True)

    def wait_scatter(b):
      pltpu.make_async_copy(rbs[b], acc.at[ebs[b].at[1]], semss[b]).wait()

    def step(k, bk, skip_scatter_wait):
      b1 = (bk + 1) % 4
      b2 = (bk + 2) % 4
      wait_idx(b1)
      start_gather(b1)
      wait_gather(bk)
      start_scatter(bk)
      if not skip_scatter_wait:
        wait_scatter(b2)
      start_idx(k + 2, b2)

    pltpu.sync_copy(eidx_hbm.at[:, pl.ds(idx_base(0), CH)], ebs[0])
    start_gather(0)
    start_idx(1, 1)
    step(jnp.int32(0), 0, True)
    step(jnp.int32(1), 1, True)

    @pl.loop(0, (nfull - 2) // 4)
    def _(p):
      for m in range(4):
        step(2 + p * 4 + m, (2 + m) % 4, False)

    wait_idx((nfull + 1) % 4)
    wait_gather(nfull % 4)
    wait_scatter((nfull - 2) % 4)
    wait_scatter((nfull - 1) % 4)
    tbase = pl.multiple_of(wid * epw + nfull * CH, 8)
    pltpu.sync_copy(eidx_hbm.at[:, pl.ds(tbase, tail)], ebt)
    pltpu.async_copy(hp_hbm.at[ebt.at[0]], rbt, semgs[0]).wait()
    pltpu.sync_copy(rbt, acc.at[ebt.at[1]], add=True)
    plsc.subcore_barrier()
    pltpu.sync_copy(acc.at[pl.ds(s * rpw, rpw), :],
                    out_hbm.at[c, pl.ds(s * rpw, rpw), :])

  return agg(hp, eidx)


def _dinv_block(d_ref):
  deg = d_ref[:, 0:1] + d_ref[:, 1:2] + jnp.float32(1.0)
  return lax.rsqrt(deg)


def _tc_layer1(x, w1, dpt):
  b = 2000
  grid = N // b

  def body(x_ref, w_ref, d_ref, o_ref):
    h = jnp.dot(x_ref[...], w_ref[...], preferred_element_type=jnp.float32)
    o_ref[...] = h * _dinv_block(d_ref)

  return pl.pallas_call(
      body,
      grid=(grid,),
      in_specs=[
          pl.BlockSpec((b, 128), lambda i: (i, 0)),
          pl.BlockSpec((128, 64), lambda i: (0, 0)),
          pl.BlockSpec((b, 2), lambda i: (i, 0)),
      ],
      out_specs=pl.BlockSpec((b, 64), lambda i: (i, 0)),
      out_shape=jax.ShapeDtypeStruct((NPAD, 64), jnp.float32),
  )(x, w1, dpt)


def _tc_layer2(ap1, hp1, dpt, b1, w2):
  b = 2000
  grid = N // b

  def body(a_ref, h_ref, d_ref, b_ref, w_ref, o_ref):
    dinv = _dinv_block(d_ref)
    t = a_ref[0] + a_ref[1] + h_ref[...]
    x2 = jnp.maximum(t * dinv + b_ref[...], jnp.float32(0.0))
    o_ref[...] = jnp.dot(x2, w_ref[...], preferred_element_type=jnp.float32) * dinv

  return pl.pallas_call(
      body,
      grid=(grid,),
      in_specs=[
          pl.BlockSpec((NC, b, 64), lambda i: (0, i, 0)),
          pl.BlockSpec((b, 64), lambda i: (i, 0)),
          pl.BlockSpec((b, 2), lambda i: (i, 0)),
          pl.BlockSpec((1, 64), lambda i: (0, 0)),
          pl.BlockSpec((64, 16), lambda i: (0, 0)),
      ],
      out_specs=pl.BlockSpec((b, 16), lambda i: (i, 0)),
      out_shape=jax.ShapeDtypeStruct((NPAD, 16), jnp.float32),
  )(ap1, hp1, dpt, b1, w2)


def _tc_final(ap2, hp2, dpt, b2):
  b = 2000
  grid = N // b

  def body(a_ref, h_ref, d_ref, b_ref, o_ref):
    t = a_ref[0] + a_ref[1] + h_ref[...]
    o_ref[...] = t * _dinv_block(d_ref) + b_ref[...]

  return pl.pallas_call(
      body,
      grid=(grid,),
      in_specs=[
          pl.BlockSpec((NC, b, 16), lambda i: (0, i, 0)),
          pl.BlockSpec((b, 16), lambda i: (i, 0)),
          pl.BlockSpec((b, 2), lambda i: (i, 0)),
          pl.BlockSpec((1, 16), lambda i: (0, 0)),
      ],
      out_specs=pl.BlockSpec((b, 16), lambda i: (i, 0)),
      out_shape=jax.ShapeDtypeStruct((N, 16), jnp.float32),
  )(ap2, hp2, dpt, b2)


def kernel(X, edge_index, W1, b1, W2, b2):
  deg_parts = _deg_hist(edge_index[1])
  dpt = jnp.transpose(deg_parts)
  hp1 = _tc_layer1(X, W1, dpt)
  ap1 = _edge_agg(hp1, edge_index, 64)
  hp2 = _tc_layer2(ap1, hp1, dpt, b1.reshape(1, 64), W2)
  ap2 = _edge_agg(hp2, edge_index, 16)
  return _tc_final(ap2, hp2, dpt, b2.reshape(1, 16))
